# Initial kernel scaffold; baseline (speedup 1.0000x reference)
import jax
import jax.numpy as jnp
from jax.experimental import pallas as pl


def kernel(tokens, table):
    raise NotImplementedError("write your pallas kernel here")


# speedup vs baseline: 3.2981x; 3.2981x over previous
import functools
import math

import jax
import jax.numpy as jnp
from jax import lax
from jax.experimental import pallas as pl
from jax.experimental.pallas import tpu as pltpu
from jax.experimental.pallas import tpu_sc as plsc

_LANES = 16
_CHUNK = 128


def _positional_encoding_host(max_len, d_model):
    position = jnp.arange(max_len, dtype=jnp.float32)[:, None]
    div_term = jnp.exp(
        jnp.arange(0, d_model, 2, dtype=jnp.float32) * (-math.log(10000.0) / d_model)
    )
    angles = position * div_term[None, :]
    pe = jnp.zeros((max_len, d_model), dtype=jnp.float32)
    pe = pe.at[:, 0::2].set(jnp.sin(angles))
    pe = pe.at[:, 1::2].set(jnp.cos(angles))
    return pe


_NBUF = 2


@functools.lru_cache(maxsize=None)
def _build_emb_kernel(batch, seq, dim, n_workers):
    n_tok = batch * seq
    tok_per_w = n_tok // n_workers
    chunks = tok_per_w // _CHUNK
    nvec = dim // _LANES
    nbuf = _NBUF
    assert chunks % nbuf == 0 and tok_per_w % seq == 0
    pe_words = (seq + _CHUNK) * dim

    mesh = plsc.VectorSubcoreMesh(core_axis_name="c", subcore_axis_name="s")

    @functools.partial(
        pl.kernel,
        out_type=jax.ShapeDtypeStruct((n_tok, dim), jnp.float32),
        mesh=mesh,
        compiler_params=pltpu.CompilerParams(use_tc_tiling_on_sc=True),
        scratch_types=[
            pltpu.VMEM((tok_per_w // _CHUNK, _CHUNK), jnp.int32),
            pltpu.VMEM((pe_words,), jnp.float32),
            [pltpu.VMEM((_CHUNK, 128), jnp.float32) for _ in range(nbuf)],
            [pltpu.VMEM((_CHUNK, dim), jnp.float32) for _ in range(nbuf)],
            [pltpu.SemaphoreType.DMA for _ in range(nbuf)],
            [pltpu.SemaphoreType.DMA for _ in range(nbuf)],
        ],
    )
    def emb(tok_ref, table_ref, pe_ref, out_ref, tok_v, pe_v, grows, crows,
            gsem, osem):
        wid = lax.axis_index("s") * 2 + lax.axis_index("c")
        tok_base = wid * tok_per_w
        pltpu.sync_copy(pe_ref, pe_v)
        pltpu.sync_copy(tok_ref.at[pl.ds(wid * chunks, chunks)], tok_v)

        def gather_start(c, b):
            pltpu.async_copy(table_ref.at[tok_v.at[c]], grows[b], gsem[b])

        def gather_wait(b):
            pltpu.make_async_copy(
                table_ref.at[pl.ds(0, _CHUNK)], grows[b], gsem[b]
            ).wait()

        def out_start(c, b):
            pltpu.async_copy(
                crows[b], out_ref.at[pl.ds(tok_base + c * _CHUNK, _CHUNK)], osem[b]
            )

        def out_wait(b):
            pltpu.make_async_copy(
                crows[b], out_ref.at[pl.ds(tok_base, _CHUNK)], osem[b]
            ).wait()

        for b in range(nbuf - 1):
            gather_start(b, b)

        def outer(go, carry):
            for b in range(nbuf):
                g = go * nbuf + b
                nxt = g + nbuf - 1
                nb = (b + nbuf - 1) % nbuf

                @pl.when(jnp.logical_and(nxt < chunks, nxt >= nbuf))
                def _():
                    out_wait(nb)

                @pl.when(nxt < chunks)
                def _():
                    gather_start(nxt, nb)

                gather_wait(b)
                pbase = ((g * _CHUNK) % seq) * dim

                def add_body(k, acc):
                    prow = pbase + k * dim
                    for j in range(nvec):
                        jo = j * _LANES
                        crows[b][k, pl.ds(jo, _LANES)] = (
                            grows[b][k, pl.ds(jo, _LANES)]
                            + pe_v[pl.ds(prow + jo, _LANES)]
                        )
                    return acc

                lax.fori_loop(0, _CHUNK, add_body, None, unroll=2)
                out_start(g, b)
            return carry

        lax.fori_loop(0, chunks // nbuf, outer, None)
        for b in range(nbuf):
            out_wait(b)

    return emb


def kernel(tokens, table):
    batch, seq = tokens.shape
    vocab, dim = table.shape
    n_workers = 32
    assert dim == 64 and (batch * seq) % (n_workers * _CHUNK) == 0

    pe = _positional_encoding_host(seq, dim)
    idx = jnp.arange(seq + _CHUNK) % seq
    pe_ext = pe[idx].reshape(-1)

    table128 = jnp.pad(table, ((0, 0), (0, 128 - dim)))
    tok128 = tokens.reshape(-1, _CHUNK).astype(jnp.int32)
    emb = _build_emb_kernel(batch, seq, dim, n_workers)
    out = emb(tok128, table128, pe_ext)
    return out.reshape(batch, seq, dim)


# speedup vs baseline: 3.4162x; 1.0358x over previous
import functools
import math

import jax
import jax.numpy as jnp
from jax import lax
from jax.experimental import pallas as pl
from jax.experimental.pallas import tpu as pltpu
from jax.experimental.pallas import tpu_sc as plsc

_LANES = 16


def _positional_encoding_host(max_len, d_model):
    position = jnp.arange(max_len, dtype=jnp.float32)[:, None]
    div_term = jnp.exp(
        jnp.arange(0, d_model, 2, dtype=jnp.float32) * (-math.log(10000.0) / d_model)
    )
    angles = position * div_term[None, :]
    pe = jnp.zeros((max_len, d_model), dtype=jnp.float32)
    pe = pe.at[:, 0::2].set(jnp.sin(angles))
    pe = pe.at[:, 1::2].set(jnp.cos(angles))
    return pe


@functools.lru_cache(maxsize=None)
def _build_emb_kernel(batch, seq, dim, n_workers):
    n_tok = batch * seq
    tok_per_w = n_tok // n_workers
    chunks = tok_per_w // seq
    half = seq // 2
    nvec = dim // _LANES
    assert chunks % 4 == 0

    mesh = plsc.VectorSubcoreMesh(core_axis_name="c", subcore_axis_name="s")

    @functools.partial(
        pl.kernel,
        out_type=jax.ShapeDtypeStruct((batch, seq, dim), jnp.float32),
        mesh=mesh,
        compiler_params=pltpu.CompilerParams(use_tc_tiling_on_sc=True),
        scratch_types=[
            [pltpu.VMEM((2, half), jnp.int32) for _ in range(4)],
            [pltpu.VMEM((seq, 128), jnp.float32) for _ in range(2)],
            [pltpu.VMEM((seq, dim), jnp.float32) for _ in range(2)],
            pltpu.VMEM((seq * dim,), jnp.float32),
            [pltpu.SemaphoreType.DMA for _ in range(4)],
            [pltpu.SemaphoreType.DMA for _ in range(2)],
            [pltpu.SemaphoreType.DMA for _ in range(2)],
        ],
    )
    def emb(tok_ref, table_ref, pe_ref, out_ref, idx_v, grows, crows, pe_v,
            isem, gsem, osem):
        wid = lax.axis_index("s") * 2 + lax.axis_index("c")
        row_base = wid * (chunks * 2)
        seq_base = wid * chunks
        pltpu.sync_copy(pe_ref, pe_v)

        def idx_start(c, i):
            pltpu.async_copy(
                tok_ref.at[pl.ds(row_base + c * 2, 2)], idx_v[i], isem[i]
            )

        def idx_wait(c, i):
            pltpu.make_async_copy(
                tok_ref.at[pl.ds(row_base + c * 2, 2)], idx_v[i], isem[i]
            ).wait()

        def gather_start(i, b):
            pltpu.async_copy(
                table_ref.at[idx_v[i].at[0]], grows[b].at[pl.ds(0, half)], gsem[b]
            )
            pltpu.async_copy(
                table_ref.at[idx_v[i].at[1]], grows[b].at[pl.ds(half, half)], gsem[b]
            )

        def gather_wait(i, b):
            pltpu.make_async_copy(
                table_ref.at[idx_v[i].at[0]], grows[b].at[pl.ds(0, half)], gsem[b]
            ).wait()
            pltpu.make_async_copy(
                table_ref.at[idx_v[i].at[1]], grows[b].at[pl.ds(half, half)], gsem[b]
            ).wait()

        def out_start(c, b):
            pltpu.async_copy(crows[b], out_ref.at[seq_base + c], osem[b])

        def out_wait(c, b):
            pltpu.make_async_copy(crows[b], out_ref.at[seq_base + c], osem[b]).wait()

        idx_start(0, 0)
        idx_start(1, 1)
        idx_wait(0, 0)
        gather_start(0, 0)

        def outer(go, carry):
            for u in range(4):
                g = go * 4 + u
                b = u % 2
                ob = 1 - b

                @pl.when(g + 2 < chunks)
                def _():
                    idx_start(g + 2, (u + 2) % 4)

                @pl.when(g + 1 < chunks)
                def _():
                    idx_wait(g + 1, (u + 1) % 4)
                    gather_start((u + 1) % 4, ob)

                @pl.when(g >= 2)
                def _():
                    out_wait(g - 2, b)

                gather_wait(u, b)

                def add_body(k, acc):
                    pb = k * dim
                    for j in range(nvec):
                        jo = j * _LANES
                        crows[b][k, pl.ds(jo, _LANES)] = (
                            grows[b][k, pl.ds(jo, _LANES)]
                            + pe_v[pl.ds(pb + jo, _LANES)]
                        )
                    return acc

                lax.fori_loop(0, seq, add_body, None, unroll=4)
                out_start(g, b)
            return carry

        lax.fori_loop(0, chunks // 4, outer, None)
        out_wait(chunks - 2, 0)
        out_wait(chunks - 1, 1)

    return emb


def kernel(tokens, table):
    batch, seq = tokens.shape
    vocab, dim = table.shape
    n_workers = 32
    assert dim == 64 and batch % n_workers == 0 and seq % 2 == 0

    pe = _positional_encoding_host(seq, dim).reshape(-1)
    table128 = jnp.pad(table, ((0, 0), (0, 128 - dim)))
    tok2d = tokens.reshape(-1, seq // 2).astype(jnp.int32)
    emb = _build_emb_kernel(batch, seq, dim, n_workers)
    return emb(tok2d, table128, pe)


# speedup vs baseline: 3.9883x; 1.1675x over previous
import functools
import math

import jax
import jax.numpy as jnp
from jax import lax
from jax.experimental import pallas as pl
from jax.experimental.pallas import tpu as pltpu
from jax.experimental.pallas import tpu_sc as plsc

_LANES = 16


def _positional_encoding_host(max_len, d_model):
    position = jnp.arange(max_len, dtype=jnp.float32)[:, None]
    div_term = jnp.exp(
        jnp.arange(0, d_model, 2, dtype=jnp.float32) * (-math.log(10000.0) / d_model)
    )
    angles = position * div_term[None, :]
    pe = jnp.zeros((max_len, d_model), dtype=jnp.float32)
    pe = pe.at[:, 0::2].set(jnp.sin(angles))
    pe = pe.at[:, 1::2].set(jnp.cos(angles))
    return pe


_NBUF = 4


@functools.lru_cache(maxsize=None)
def _build_emb_kernel(batch, seq, dim, n_workers):
    n_tok = batch * seq
    tok_per_w = n_tok // n_workers
    chunks = tok_per_w // seq
    half = seq // 2
    nvec = dim // _LANES
    nbuf = _NBUF
    assert chunks % nbuf == 0

    mesh = plsc.VectorSubcoreMesh(core_axis_name="c", subcore_axis_name="s")

    @functools.partial(
        pl.kernel,
        out_type=jax.ShapeDtypeStruct((batch, seq, dim), jnp.float32),
        mesh=mesh,
        compiler_params=pltpu.CompilerParams(use_tc_tiling_on_sc=False),
        scratch_types=[
            pltpu.VMEM((2 * chunks, half), jnp.int32),
            [pltpu.VMEM((seq, dim), jnp.float32) for _ in range(nbuf)],
            pltpu.VMEM((seq, dim), jnp.float32),
            [pltpu.SemaphoreType.DMA for _ in range(nbuf)],
            [pltpu.SemaphoreType.DMA for _ in range(nbuf)],
        ],
    )
    def emb(tok_ref, table_ref, pe_ref, out_ref, idx_v, rows_v, pe_v, gsem, osem):
        wid = lax.axis_index("s") * 2 + lax.axis_index("c")
        pltpu.sync_copy(pe_ref, pe_v)
        row_base = wid * (chunks * 2)
        seq_base = wid * chunks
        pltpu.sync_copy(tok_ref.at[pl.ds(row_base, 2 * chunks)], idx_v)

        def gather_start(c, b):
            r = c * 2
            pltpu.async_copy(
                table_ref.at[idx_v.at[r]], rows_v[b].at[pl.ds(0, half)], gsem[b]
            )
            pltpu.async_copy(
                table_ref.at[idx_v.at[r + 1]], rows_v[b].at[pl.ds(half, half)], gsem[b]
            )

        def gather_wait(b):
            pltpu.make_async_copy(out_ref.at[seq_base], rows_v[b], gsem[b]).wait()

        def out_start(c, b):
            pltpu.async_copy(rows_v[b], out_ref.at[seq_base + c], osem[b])

        def out_wait(b):
            pltpu.make_async_copy(rows_v[b], out_ref.at[seq_base], osem[b]).wait()

        for b in range(nbuf - 1):
            gather_start(b, b)

        def outer(go, carry):
            for b in range(nbuf):
                g = go * nbuf + b
                nxt = g + nbuf - 1
                nb = (b + nbuf - 1) % nbuf

                @pl.when(jnp.logical_and(nxt < chunks, nxt >= nbuf))
                def _():
                    out_wait(nb)

                @pl.when(nxt < chunks)
                def _():
                    gather_start(nxt, nb)

                gather_wait(b)

                def add_body(i, acc):
                    for j in range(nvec):
                        sl = pl.ds(j * _LANES, _LANES)
                        plsc.addupdate(rows_v[b].at[i, sl], pe_v[i, sl])
                    return acc

                lax.fori_loop(0, seq, add_body, None, unroll=4)
                out_start(g, b)
            return carry

        lax.fori_loop(0, chunks // nbuf, outer, None)
        for b in range(nbuf):
            out_wait(b)

    return emb


def kernel(tokens, table):
    batch, seq = tokens.shape
    vocab, dim = table.shape
    n_tok = batch * seq
    n_workers = 32
    assert n_tok % (n_workers * seq) == 0 and seq % 2 == 0 and dim % _LANES == 0

    pe = _positional_encoding_host(seq, dim)
    tok2d = tokens.reshape(-1, seq // 2).astype(jnp.int32)
    emb = _build_emb_kernel(batch, seq, dim, n_workers)
    return emb(tok2d, table, pe)
